# baseline (speedup 1.0000x reference)
import functools

import jax
import jax.numpy as jnp
from jax import lax
from jax.experimental import pallas as pl
from jax.experimental.pallas import tpu as pltpu
from jax.experimental.pallas import tpu_sc as plsc

VOCAB = 1000000
EMBED_DIM = 64
NUM_CLASS = 1000
BATCH = 4096
SEQ = 200

NUM_CORES = 2
NUM_SUBCORES = 16
NUM_WORKERS = NUM_CORES * NUM_SUBCORES
B_PER_W = BATCH // NUM_WORKERS
HALF_SEQ = SEQ // 2


def _pool_body(ids_hbm, table_hbm, out_hbm, idx_v, gbuf, pooled_v, sem):
    wid = lax.axis_index("c") * NUM_SUBCORES + lax.axis_index("s")
    pltpu.sync_copy(ids_hbm.at[wid], idx_v)

    inv_seq = jnp.float32(1.0 / SEQ)

    def row_body(r, carry):
        c0 = pltpu.async_copy(
            table_hbm.at[idx_v.at[r, 0]], gbuf.at[pl.ds(0, HALF_SEQ)], sem)
        c1 = pltpu.async_copy(
            table_hbm.at[idx_v.at[r, 1]], gbuf.at[pl.ds(HALF_SEQ, HALF_SEQ)], sem)
        c0.wait()
        c1.wait()

        def red_body(j, accs):
            a0, a1, a2, a3 = accs
            a0 = a0 + gbuf[j, pl.ds(0, 16)]
            a1 = a1 + gbuf[j, pl.ds(16, 16)]
            a2 = a2 + gbuf[j, pl.ds(32, 16)]
            a3 = a3 + gbuf[j, pl.ds(48, 16)]
            return (a0, a1, a2, a3)

        zero = jnp.zeros((16,), jnp.float32)
        a0, a1, a2, a3 = lax.fori_loop(
            0, SEQ, red_body, (zero, zero, zero, zero))
        pooled_v[r, pl.ds(0, 16)] = a0 * inv_seq
        pooled_v[r, pl.ds(16, 16)] = a1 * inv_seq
        pooled_v[r, pl.ds(32, 16)] = a2 * inv_seq
        pooled_v[r, pl.ds(48, 16)] = a3 * inv_seq
        return carry

    lax.fori_loop(0, B_PER_W, row_body, 0)
    pltpu.sync_copy(pooled_v, out_hbm.at[wid])


def _sc_pool(input_ids, emb_table):
    ids = input_ids.reshape(NUM_WORKERS, B_PER_W, 2, HALF_SEQ)
    mesh = plsc.VectorSubcoreMesh(core_axis_name="c", subcore_axis_name="s")
    f = pl.kernel(
        _pool_body,
        out_type=jax.ShapeDtypeStruct((NUM_WORKERS, B_PER_W, EMBED_DIM),
                                      jnp.float32),
        mesh=mesh,
        scratch_types=[
            pltpu.VMEM((B_PER_W, 2, HALF_SEQ), jnp.int32),
            pltpu.VMEM((SEQ, EMBED_DIM), jnp.float32),
            pltpu.VMEM((B_PER_W, EMBED_DIM), jnp.float32),
            pltpu.SemaphoreType.DMA,
        ],
        compiler_params=pltpu.CompilerParams(use_tc_tiling_on_sc=False),
    )
    pooled = f(ids, emb_table)
    return pooled.reshape(BATCH, EMBED_DIM)


BM = 256


def _matmul_body(p_ref, w_ref, b_ref, o_ref):
    acc = lax.dot_general(
        p_ref[...], w_ref[...],
        dimension_numbers=(((1,), (1,)), ((), ())),
        preferred_element_type=jnp.float32)
    o_ref[...] = acc + b_ref[...]


def _tc_head(pooled, fc_w, fc_b):
    bias = fc_b.reshape(1, NUM_CLASS)
    return pl.pallas_call(
        _matmul_body,
        grid=(BATCH // BM,),
        in_specs=[
            pl.BlockSpec((BM, EMBED_DIM), lambda i: (i, 0)),
            pl.BlockSpec((NUM_CLASS, EMBED_DIM), lambda i: (0, 0)),
            pl.BlockSpec((1, NUM_CLASS), lambda i: (0, 0)),
        ],
        out_specs=pl.BlockSpec((BM, NUM_CLASS), lambda i: (i, 0)),
        out_shape=jax.ShapeDtypeStruct((BATCH, NUM_CLASS), jnp.float32),
    )(pooled, fc_w, bias)


def kernel(input_ids, emb_table, fc_w, fc_b):
    pooled = _sc_pool(input_ids, emb_table)
    return _tc_head(pooled, fc_w, fc_b)


# speedup vs baseline: 1.2280x; 1.2280x over previous
import functools

import jax
import jax.numpy as jnp
from jax import lax
from jax.experimental import pallas as pl
from jax.experimental.pallas import tpu as pltpu
from jax.experimental.pallas import tpu_sc as plsc

VOCAB = 1000000
EMBED_DIM = 64
NUM_CLASS = 1000
BATCH = 4096
SEQ = 200

NUM_CORES = 2
NUM_SUBCORES = 16
NUM_WORKERS = NUM_CORES * NUM_SUBCORES
B_PER_W = BATCH // NUM_WORKERS
HALF_SEQ = SEQ // 2


NBUF = 4
UNROLL = 8


def _pool_body(ids_hbm, table_hbm, out_hbm, idx_v, gbuf, pooled_v, sems):
    wid = lax.axis_index("c") * NUM_SUBCORES + lax.axis_index("s")
    pltpu.sync_copy(ids_hbm.at[wid], idx_v)

    inv_seq = jnp.float32(1.0 / SEQ)

    def start_gather(r, b):
        pltpu.async_copy(
            table_hbm.at[idx_v.at[r, 0]],
            gbuf.at[b, pl.ds(0, HALF_SEQ)], sems.at[b])
        pltpu.async_copy(
            table_hbm.at[idx_v.at[r, 1]],
            gbuf.at[b, pl.ds(HALF_SEQ, HALF_SEQ)], sems.at[b])

    def wait_gather(b):
        pltpu.make_async_copy(
            table_hbm.at[idx_v.at[0, 0]],
            gbuf.at[b, pl.ds(0, HALF_SEQ)], sems.at[b]).wait()
        pltpu.make_async_copy(
            table_hbm.at[idx_v.at[0, 1]],
            gbuf.at[b, pl.ds(HALF_SEQ, HALF_SEQ)], sems.at[b]).wait()

    def reduce_slot(r, b):
        def red_body(j, accs):
            accs = list(accs)
            for u in range(UNROLL):
                row = j * UNROLL + u
                for k in range(4):
                    a = u % 2 + 2 * k
                    accs[a] = accs[a] + gbuf[b, row, pl.ds(16 * k, 16)]
            return tuple(accs)

        zero = jnp.zeros((16,), jnp.float32)
        accs = lax.fori_loop(0, SEQ // UNROLL, red_body, (zero,) * 8)
        for k in range(4):
            pooled_v[r, pl.ds(16 * k, 16)] = (
                (accs[2 * k] + accs[2 * k + 1]) * inv_seq)

    for b in range(NBUF):
        start_gather(b, b)

    def outer(g, carry):
        for b in range(NBUF):
            r = g * NBUF + b
            wait_gather(b)
            reduce_slot(r, b)

            @pl.when(r + NBUF < B_PER_W)
            def _():
                start_gather(r + NBUF, b)
        return carry

    lax.fori_loop(0, B_PER_W // NBUF, outer, 0)
    pltpu.sync_copy(pooled_v, out_hbm.at[wid])


def _sc_pool(input_ids, emb_table):
    ids = input_ids.reshape(NUM_WORKERS, B_PER_W, 2, HALF_SEQ)
    mesh = plsc.VectorSubcoreMesh(core_axis_name="c", subcore_axis_name="s")
    f = pl.kernel(
        _pool_body,
        out_type=jax.ShapeDtypeStruct((NUM_WORKERS, B_PER_W, EMBED_DIM),
                                      jnp.float32),
        mesh=mesh,
        scratch_types=[
            pltpu.VMEM((B_PER_W, 2, HALF_SEQ), jnp.int32),
            pltpu.VMEM((NBUF, SEQ, EMBED_DIM), jnp.float32),
            pltpu.VMEM((B_PER_W, EMBED_DIM), jnp.float32),
            pltpu.SemaphoreType.DMA((NBUF,)),
        ],
        compiler_params=pltpu.CompilerParams(use_tc_tiling_on_sc=False),
    )
    pooled = f(ids, emb_table)
    return pooled.reshape(BATCH, EMBED_DIM)


BM = 256


def _matmul_body(p_ref, w_ref, b_ref, o_ref):
    acc = lax.dot_general(
        p_ref[...], w_ref[...],
        dimension_numbers=(((1,), (1,)), ((), ())),
        preferred_element_type=jnp.float32)
    o_ref[...] = acc + b_ref[...]


def _tc_head(pooled, fc_w, fc_b):
    bias = fc_b.reshape(1, NUM_CLASS)
    return pl.pallas_call(
        _matmul_body,
        grid=(BATCH // BM,),
        in_specs=[
            pl.BlockSpec((BM, EMBED_DIM), lambda i: (i, 0)),
            pl.BlockSpec((NUM_CLASS, EMBED_DIM), lambda i: (0, 0)),
            pl.BlockSpec((1, NUM_CLASS), lambda i: (0, 0)),
        ],
        out_specs=pl.BlockSpec((BM, NUM_CLASS), lambda i: (i, 0)),
        out_shape=jax.ShapeDtypeStruct((BATCH, NUM_CLASS), jnp.float32),
    )(pooled, fc_w, bias)


def kernel(input_ids, emb_table, fc_w, fc_b):
    pooled = _sc_pool(input_ids, emb_table)
    return _tc_head(pooled, fc_w, fc_b)


# speedup vs baseline: 1.2337x; 1.0046x over previous
import functools

import jax
import jax.numpy as jnp
from jax import lax
from jax.experimental import pallas as pl
from jax.experimental.pallas import tpu as pltpu
from jax.experimental.pallas import tpu_sc as plsc

VOCAB = 1000000
EMBED_DIM = 64
NUM_CLASS = 1000
BATCH = 4096
SEQ = 200

NUM_CORES = 2
NUM_SUBCORES = 16
NUM_WORKERS = NUM_CORES * NUM_SUBCORES
B_PER_W = BATCH // NUM_WORKERS
S0 = 128
S1 = SEQ - S0


NBUF = 4
UNROLL = 8


def _pool_body(ids_hbm, table_hbm, out_hbm, idx_v, gbuf, pooled_v, sems):
    wid = lax.axis_index("c") * NUM_SUBCORES + lax.axis_index("s")
    base = wid * B_PER_W
    pltpu.sync_copy(ids_hbm.at[pl.ds(base, B_PER_W), :], idx_v)

    inv_seq = jnp.float32(1.0 / SEQ)

    def start_gather(r, b):
        pltpu.async_copy(
            table_hbm.at[idx_v.at[r, pl.ds(0, S0)]],
            gbuf.at[b, pl.ds(0, S0)], sems.at[b])
        pltpu.async_copy(
            table_hbm.at[idx_v.at[r, pl.ds(S0, S1)]],
            gbuf.at[b, pl.ds(S0, S1)], sems.at[b])

    def wait_gather(b):
        pltpu.make_async_copy(
            table_hbm.at[idx_v.at[0, pl.ds(0, S0)]],
            gbuf.at[b, pl.ds(0, S0)], sems.at[b]).wait()
        pltpu.make_async_copy(
            table_hbm.at[idx_v.at[0, pl.ds(S0, S1)]],
            gbuf.at[b, pl.ds(S0, S1)], sems.at[b]).wait()

    def reduce_slot(r, b):
        def red_body(j, accs):
            accs = list(accs)
            for u in range(UNROLL):
                row = j * UNROLL + u
                for k in range(4):
                    a = u % 2 + 2 * k
                    accs[a] = accs[a] + gbuf[b, row, pl.ds(16 * k, 16)]
            return tuple(accs)

        zero = jnp.zeros((16,), jnp.float32)
        accs = lax.fori_loop(0, SEQ // UNROLL, red_body, (zero,) * 8)
        for k in range(4):
            pooled_v[r, pl.ds(16 * k, 16)] = (
                (accs[2 * k] + accs[2 * k + 1]) * inv_seq)

    for b in range(NBUF):
        start_gather(b, b)

    def outer(g, carry):
        for b in range(NBUF):
            r = g * NBUF + b
            wait_gather(b)
            reduce_slot(r, b)

            @pl.when(r + NBUF < B_PER_W)
            def _():
                start_gather(r + NBUF, b)
        return carry

    lax.fori_loop(0, B_PER_W // NBUF, outer, 0)
    pltpu.sync_copy(pooled_v, out_hbm.at[pl.ds(base, B_PER_W), :])


def _sc_pool(input_ids, emb_table):
    mesh = plsc.VectorSubcoreMesh(core_axis_name="c", subcore_axis_name="s")
    f = pl.kernel(
        _pool_body,
        out_type=jax.ShapeDtypeStruct((BATCH, EMBED_DIM), jnp.float32),
        mesh=mesh,
        scratch_types=[
            pltpu.VMEM((B_PER_W, SEQ), jnp.int32),
            pltpu.VMEM((NBUF, SEQ, EMBED_DIM), jnp.float32),
            pltpu.VMEM((B_PER_W, EMBED_DIM), jnp.float32),
            pltpu.SemaphoreType.DMA((NBUF,)),
        ],
        compiler_params=pltpu.CompilerParams(use_tc_tiling_on_sc=False),
    )
    return f(input_ids, emb_table)


BM = 256


def _matmul_body(p_ref, w_ref, b_ref, o_ref):
    acc = lax.dot_general(
        p_ref[...], w_ref[...],
        dimension_numbers=(((1,), (1,)), ((), ())),
        preferred_element_type=jnp.float32)
    o_ref[...] = acc + b_ref[...]


def _tc_head(pooled, fc_w, fc_b):
    bias = fc_b.reshape(1, NUM_CLASS)
    return pl.pallas_call(
        _matmul_body,
        grid=(BATCH // BM,),
        in_specs=[
            pl.BlockSpec((BM, EMBED_DIM), lambda i: (i, 0)),
            pl.BlockSpec((NUM_CLASS, EMBED_DIM), lambda i: (0, 0)),
            pl.BlockSpec((1, NUM_CLASS), lambda i: (0, 0)),
        ],
        out_specs=pl.BlockSpec((BM, NUM_CLASS), lambda i: (i, 0)),
        out_shape=jax.ShapeDtypeStruct((BATCH, NUM_CLASS), jnp.float32),
    )(pooled, fc_w, bias)


def kernel(input_ids, emb_table, fc_w, fc_b):
    pooled = _sc_pool(input_ids, emb_table)
    return _tc_head(pooled, fc_w, fc_b)
